# Initial kernel scaffold; baseline (speedup 1.0000x reference)
import jax
import jax.numpy as jnp
from jax.experimental import pallas as pl


def kernel(x, rms_w, Wg, bg, W1a, b1a, W1b, b1b, W2, b2):
    raise NotImplementedError("write your pallas kernel here")


# speedup vs baseline: 1.0316x; 1.0316x over previous
import functools

import jax
import jax.numpy as jnp
from jax.experimental import pallas as pl
from jax.experimental.pallas import tpu as pltpu

EPS = 1.1920929e-07


def _moe_body(x_ref, rms_ref, wg_ref, bg_ref,
              w1a_ref, b1a_ref, w1b_ref, b1b_ref, w2_ref, b2_ref,
              out_ref, w_out_ref, xn_ref, *, n_e):
    e = pl.program_id(1)
    f = pl.program_id(2)

    @pl.when((e == 0) & (f == 0))
    def _gate():
        xv = x_ref[...]
        inv = jax.lax.rsqrt(jnp.mean(xv * xv, axis=-1, keepdims=True) + EPS)
        xn = xv * inv * rms_ref[...]
        xn_ref[...] = xn
        logits = jnp.dot(xn, wg_ref[...],
                         preferred_element_type=jnp.float32) + bg_ref[...]
        iota = jax.lax.broadcasted_iota(jnp.int32, logits.shape, 1)
        m1 = jnp.max(logits, axis=-1, keepdims=True)
        idx1 = jnp.min(jnp.where(logits == m1, iota, n_e),
                       axis=-1, keepdims=True)
        masked = jnp.where(iota == idx1, -jnp.inf, logits)
        m2 = jnp.max(masked, axis=-1, keepdims=True)
        idx2 = jnp.min(jnp.where(masked == m2, iota, n_e),
                       axis=-1, keepdims=True)
        z = jnp.exp(m2 - m1)
        denom = 1.0 + z
        w1 = 1.0 / denom
        w2 = z / denom
        w_pe = (jnp.where(iota == idx1, w1, 0.0)
                + jnp.where(iota == idx2, w2, 0.0))
        w_out_ref[...] = w_pe
        out_ref[...] = jnp.zeros_like(out_ref)

    xn = xn_ref[...]
    a = jnp.dot(xn, w1a_ref[0], preferred_element_type=jnp.float32) \
        + b1a_ref[0]
    b = jnp.dot(xn, w1b_ref[0], preferred_element_type=jnp.float32) \
        + b1b_ref[0]
    h = (a * jax.nn.sigmoid(a)) * b
    eo = jnp.dot(h, w2_ref[0], preferred_element_type=jnp.float32)
    iota = jax.lax.broadcasted_iota(jnp.int32, w_out_ref.shape, 1)
    we = jnp.sum(jnp.where(iota == e, w_out_ref[...], 0.0),
                 axis=-1, keepdims=True)
    eo = jnp.where(f == 0, eo + b2_ref[0], eo)
    out_ref[...] += we * eo


@functools.partial(jax.jit, static_argnames=("interpret",))
def kernel(x, rms_w, Wg, bg, W1a, b1a, W1b, b1b, W2, b2, interpret=False):
    B, S, DIM = x.shape
    E = Wg.shape[1]
    DFF = W1a.shape[2]
    N = B * S
    TT = min(512, N)
    FB = min(1024, DFF)
    nt, nf = N // TT, DFF // FB

    xf = x.reshape(N, DIM)
    rms2 = rms_w.reshape(1, DIM)
    bg2 = bg.reshape(1, E)
    b1a3 = b1a.reshape(E, 1, DFF)
    b1b3 = b1b.reshape(E, 1, DFF)
    b23 = b2.reshape(E, 1, DIM)

    out, w_pe = pl.pallas_call(
        functools.partial(_moe_body, n_e=E),
        grid=(nt, E, nf),
        in_specs=[
            pl.BlockSpec((TT, DIM), lambda t, e, f: (t, 0)),
            pl.BlockSpec((1, DIM), lambda t, e, f: (0, 0)),
            pl.BlockSpec((DIM, E), lambda t, e, f: (0, 0)),
            pl.BlockSpec((1, E), lambda t, e, f: (0, 0)),
            pl.BlockSpec((1, DIM, FB), lambda t, e, f: (e, 0, f)),
            pl.BlockSpec((1, 1, FB), lambda t, e, f: (e, 0, f)),
            pl.BlockSpec((1, DIM, FB), lambda t, e, f: (e, 0, f)),
            pl.BlockSpec((1, 1, FB), lambda t, e, f: (e, 0, f)),
            pl.BlockSpec((1, FB, DIM), lambda t, e, f: (e, f, 0)),
            pl.BlockSpec((1, 1, DIM), lambda t, e, f: (e, 0, 0)),
        ],
        out_specs=[
            pl.BlockSpec((TT, DIM), lambda t, e, f: (t, 0)),
            pl.BlockSpec((TT, E), lambda t, e, f: (t, 0)),
        ],
        out_shape=[
            jax.ShapeDtypeStruct((N, DIM), jnp.float32),
            jax.ShapeDtypeStruct((N, E), jnp.float32),
        ],
        scratch_shapes=[pltpu.VMEM((TT, DIM), jnp.float32)],
        interpret=interpret,
    )(xf, rms2, Wg, bg2, W1a, b1a3, W1b, b1b3, W2, b23)

    return out.reshape(B, S, DIM), w_pe.reshape(B, S, E)


# speedup vs baseline: 1.0842x; 1.0511x over previous
import functools

import jax
import jax.numpy as jnp
from jax.experimental import pallas as pl
from jax.experimental.pallas import tpu as pltpu

EPS = 1.1920929e-07


def _moe_body(x_ref, rms_ref, wg_ref, bg_ref,
              w1a_ref, b1a_ref, w1b_ref, b1b_ref, w2_ref, b2_ref,
              out_ref, w_out_ref, xn_ref, *, n_e):
    e = pl.program_id(1)
    f = pl.program_id(2)

    @pl.when((e == 0) & (f == 0))
    def _gate():
        xv = x_ref[...]
        inv = jax.lax.rsqrt(jnp.mean(xv * xv, axis=-1, keepdims=True) + EPS)
        xn = xv * inv * rms_ref[...]
        xn_ref[...] = xn.astype(jnp.bfloat16)
        logits = jnp.dot(xn, wg_ref[...],
                         preferred_element_type=jnp.float32) + bg_ref[...]
        iota = jax.lax.broadcasted_iota(jnp.int32, logits.shape, 1)
        m1 = jnp.max(logits, axis=-1, keepdims=True)
        idx1 = jnp.min(jnp.where(logits == m1, iota, n_e),
                       axis=-1, keepdims=True)
        masked = jnp.where(iota == idx1, -jnp.inf, logits)
        m2 = jnp.max(masked, axis=-1, keepdims=True)
        idx2 = jnp.min(jnp.where(masked == m2, iota, n_e),
                       axis=-1, keepdims=True)
        z = jnp.exp(m2 - m1)
        denom = 1.0 + z
        w1 = 1.0 / denom
        w2 = z / denom
        w_pe = (jnp.where(iota == idx1, w1, 0.0)
                + jnp.where(iota == idx2, w2, 0.0))
        w_out_ref[...] = w_pe
        out_ref[...] = jnp.zeros_like(out_ref)

    xn = xn_ref[...]
    a = jnp.dot(xn, w1a_ref[0], preferred_element_type=jnp.float32) \
        + b1a_ref[0]
    b = jnp.dot(xn, w1b_ref[0], preferred_element_type=jnp.float32) \
        + b1b_ref[0]
    h = (a * jax.nn.sigmoid(a)) * b
    eo = jnp.dot(h.astype(jnp.bfloat16), w2_ref[0],
                 preferred_element_type=jnp.float32)
    iota = jax.lax.broadcasted_iota(jnp.int32, w_out_ref.shape, 1)
    we = jnp.sum(jnp.where(iota == e, w_out_ref[...], 0.0),
                 axis=-1, keepdims=True)
    eo = jnp.where(f == 0, eo + b2_ref[0], eo)
    out_ref[...] += we * eo


@functools.partial(jax.jit, static_argnames=("interpret",))
def kernel(x, rms_w, Wg, bg, W1a, b1a, W1b, b1b, W2, b2, interpret=False):
    B, S, DIM = x.shape
    E = Wg.shape[1]
    DFF = W1a.shape[2]
    N = B * S
    TT = min(512, N)
    FB = min(1024, DFF)
    nt, nf = N // TT, DFF // FB

    xf = x.reshape(N, DIM)
    rms2 = rms_w.reshape(1, DIM)
    bg2 = bg.reshape(1, E)
    b1a3 = b1a.reshape(E, 1, DFF)
    b1b3 = b1b.reshape(E, 1, DFF)
    b23 = b2.reshape(E, 1, DIM)
    W1a = W1a.astype(jnp.bfloat16)
    W1b = W1b.astype(jnp.bfloat16)
    W2 = W2.astype(jnp.bfloat16)

    out, w_pe = pl.pallas_call(
        functools.partial(_moe_body, n_e=E),
        grid=(nt, E, nf),
        in_specs=[
            pl.BlockSpec((TT, DIM), lambda t, e, f: (t, 0)),
            pl.BlockSpec((1, DIM), lambda t, e, f: (0, 0)),
            pl.BlockSpec((DIM, E), lambda t, e, f: (0, 0)),
            pl.BlockSpec((1, E), lambda t, e, f: (0, 0)),
            pl.BlockSpec((1, DIM, FB), lambda t, e, f: (e, 0, f)),
            pl.BlockSpec((1, 1, FB), lambda t, e, f: (e, 0, f)),
            pl.BlockSpec((1, DIM, FB), lambda t, e, f: (e, 0, f)),
            pl.BlockSpec((1, 1, FB), lambda t, e, f: (e, 0, f)),
            pl.BlockSpec((1, FB, DIM), lambda t, e, f: (e, f, 0)),
            pl.BlockSpec((1, 1, DIM), lambda t, e, f: (e, 0, 0)),
        ],
        out_specs=[
            pl.BlockSpec((TT, DIM), lambda t, e, f: (t, 0)),
            pl.BlockSpec((TT, E), lambda t, e, f: (t, 0)),
        ],
        out_shape=[
            jax.ShapeDtypeStruct((N, DIM), jnp.float32),
            jax.ShapeDtypeStruct((N, E), jnp.float32),
        ],
        scratch_shapes=[pltpu.VMEM((TT, DIM), jnp.bfloat16)],
        interpret=interpret,
    )(xf, rms2, Wg, bg2, W1a, b1a3, W1b, b1b3, W2, b23)

    return out.reshape(B, S, DIM), w_pe.reshape(B, S, E)


# speedup vs baseline: 1.1331x; 1.0451x over previous
import functools

import jax
import jax.numpy as jnp
from jax.experimental import pallas as pl
from jax.experimental.pallas import tpu as pltpu

EPS = 1.1920929e-07


def _moe_body(x_ref, rms_ref, wg_ref, bg_ref,
              w1a_ref, b1a_ref, w1b_ref, b1b_ref, w2_ref, b2_ref,
              out_ref, w_out_ref, xn_ref, *, n_e):
    e = pl.program_id(1)
    f = pl.program_id(2)

    @pl.when((e == 0) & (f == 0))
    def _gate():
        xv = x_ref[...]
        inv = jax.lax.rsqrt(jnp.mean(xv * xv, axis=-1, keepdims=True) + EPS)
        xn = xv * inv * rms_ref[...]
        xn_ref[...] = xn.astype(jnp.bfloat16)
        logits = jnp.dot(xn, wg_ref[...],
                         preferred_element_type=jnp.float32) + bg_ref[...]
        iota = jax.lax.broadcasted_iota(jnp.int32, logits.shape, 1)
        m1 = jnp.max(logits, axis=-1, keepdims=True)
        idx1 = jnp.min(jnp.where(logits == m1, iota, n_e),
                       axis=-1, keepdims=True)
        masked = jnp.where(iota == idx1, -jnp.inf, logits)
        m2 = jnp.max(masked, axis=-1, keepdims=True)
        idx2 = jnp.min(jnp.where(masked == m2, iota, n_e),
                       axis=-1, keepdims=True)
        z = jnp.exp(m2 - m1)
        denom = 1.0 + z
        w1 = 1.0 / denom
        w2 = z / denom
        w_pe = (jnp.where(iota == idx1, w1, 0.0)
                + jnp.where(iota == idx2, w2, 0.0))
        w_out_ref[...] = w_pe
        out_ref[...] = jnp.zeros_like(out_ref)

    xn = xn_ref[...]
    a = jnp.dot(xn, w1a_ref[0], preferred_element_type=jnp.float32) \
        + b1a_ref[0]
    b = jnp.dot(xn, w1b_ref[0], preferred_element_type=jnp.float32) \
        + b1b_ref[0]
    h = ((a * jax.nn.sigmoid(a)) * b).astype(jnp.bfloat16)
    eo = jnp.dot(h, w2_ref[0], preferred_element_type=jnp.float32)
    iota = jax.lax.broadcasted_iota(jnp.int32, w_out_ref.shape, 1)
    we = jnp.sum(jnp.where(iota == e, w_out_ref[...], 0.0),
                 axis=-1, keepdims=True)
    eo = jnp.where(f == 0, eo + b2_ref[0], eo)
    out_ref[...] += we * eo


@functools.partial(jax.jit, static_argnames=("interpret",))
def kernel(x, rms_w, Wg, bg, W1a, b1a, W1b, b1b, W2, b2, interpret=False):
    B, S, DIM = x.shape
    E = Wg.shape[1]
    DFF = W1a.shape[2]
    N = B * S
    TT = min(1024, N)
    FB = min(1024, DFF)
    nt, nf = N // TT, DFF // FB

    xf = x.reshape(N, DIM)
    rms2 = rms_w.reshape(1, DIM)
    bg2 = bg.reshape(1, E)
    b1a3 = b1a.reshape(E, 1, DFF)
    b1b3 = b1b.reshape(E, 1, DFF)
    b23 = b2.reshape(E, 1, DIM)
    W1a = W1a.astype(jnp.bfloat16)
    W1b = W1b.astype(jnp.bfloat16)
    W2 = W2.astype(jnp.bfloat16)

    out, w_pe = pl.pallas_call(
        functools.partial(_moe_body, n_e=E),
        grid=(nt, E, nf),
        in_specs=[
            pl.BlockSpec((TT, DIM), lambda t, e, f: (t, 0)),
            pl.BlockSpec((1, DIM), lambda t, e, f: (0, 0)),
            pl.BlockSpec((DIM, E), lambda t, e, f: (0, 0)),
            pl.BlockSpec((1, E), lambda t, e, f: (0, 0)),
            pl.BlockSpec((1, DIM, FB), lambda t, e, f: (e, 0, f)),
            pl.BlockSpec((1, 1, FB), lambda t, e, f: (e, 0, f)),
            pl.BlockSpec((1, DIM, FB), lambda t, e, f: (e, 0, f)),
            pl.BlockSpec((1, 1, FB), lambda t, e, f: (e, 0, f)),
            pl.BlockSpec((1, FB, DIM), lambda t, e, f: (e, f, 0)),
            pl.BlockSpec((1, 1, DIM), lambda t, e, f: (e, 0, 0)),
        ],
        out_specs=[
            pl.BlockSpec((TT, DIM), lambda t, e, f: (t, 0)),
            pl.BlockSpec((TT, E), lambda t, e, f: (t, 0)),
        ],
        out_shape=[
            jax.ShapeDtypeStruct((N, DIM), jnp.float32),
            jax.ShapeDtypeStruct((N, E), jnp.float32),
        ],
        scratch_shapes=[pltpu.VMEM((TT, DIM), jnp.bfloat16)],
        interpret=interpret,
    )(xf, rms2, Wg, bg2, W1a, b1a3, W1b, b1b3, W2, b23)

    return out.reshape(B, S, DIM), w_pe.reshape(B, S, E)
